# baseline (speedup 1.0000x reference)
import functools
import math

import jax
import jax.numpy as jnp
from jax import lax
from jax.experimental import pallas as pl
from jax.experimental.pallas import tpu as pltpu
from jax.experimental.pallas import tpu_sc as plsc

D = 64
L = 200
B = 4096
ROWS = B * L
NW = 32
RPW = ROWS // NW
CHUNK = 128
NCHUNK = RPW // CHUNK
NBUF = 4
SCALE = math.sqrt(D)
CB = CHUNK * D * 4


def _pe_kernel(out_ref):
    pos = lax.broadcasted_iota(jnp.int32, (L, D), 0).astype(jnp.float32)
    col = lax.broadcasted_iota(jnp.int32, (L, D), 1)
    even = col - (col % 2)
    div = jnp.exp(even.astype(jnp.float32) * (-math.log(10000.0) / D))
    ang = pos * div
    out_ref[...] = jnp.where(col % 2 == 0, jnp.sin(ang), jnp.cos(ang))


def _pe_table():
    return pl.pallas_call(
        _pe_kernel,
        out_shape=jax.ShapeDtypeStruct((L, D), jnp.float32),
    )()


def _sc_kernel(idx_hbm, table_hbm, pe_hbm, out_hbm,
               pe_v, i0, i1, i2, i3, r0, r1, r2, r3,
               g0, g1, g2, g3, o0, o1, o2, o3):
    idxv = [i0, i1, i2, i3]
    rows = [r0, r1, r2, r3]
    gsem = [g0, g1, g2, g3]
    osem = [o0, o1, o2, o3]

    wid = lax.axis_index("s") * 2 + lax.axis_index("c")
    base = wid * RPW
    ibase = wid * NCHUNK

    pltpu.sync_copy(pe_hbm, pe_v)

    def fire(c, b):
        pltpu.sync_copy(idx_hbm.at[pl.ds(ibase + c, 1)], idxv[b])
        pltpu.async_copy(table_hbm.at[idxv[b].at[0]], rows[b], gsem[b])

    def drain(sem, b):
        pltpu.make_async_copy(out_hbm.at[pl.ds(0, CHUNK)], rows[b], sem).wait()

    for b in range(NBUF - 1):
        fire(b, b)

    def body(g, _):
        for b in range(NBUF):
            c = g * NBUF + b
            bb = (b + NBUF - 1) % NBUF

            if b == 0:
                @pl.when(g > 0)
                def _():
                    drain(osem[bb], bb)
                fire(c + NBUF - 1, bb)
            else:
                @pl.when(g < NCHUNK // NBUF - 1)
                def _():
                    drain(osem[bb], bb)
                    fire(c + NBUF - 1, bb)

            drain(gsem[b], b)

            start = lax.rem(c * CHUNK, L)

            def crow(r, _, b=b):
                pos = start + r
                pos = jnp.where(pos >= L, pos - L, pos)
                for j in range(D // 16):
                    sl = pl.ds(j * 16, 16)
                    rows[b][r, sl] = rows[b][r, sl] * SCALE + pe_v[pos, sl]
                return ()

            lax.fori_loop(0, CHUNK, crow, ())

            pltpu.async_copy(rows[b], out_hbm.at[pl.ds(base + c * CHUNK, CHUNK)],
                             osem[b])
        return ()

    lax.fori_loop(0, NCHUNK // NBUF, body, ())

    for b in range(NBUF):
        drain(osem[b], b)


def _make_sc_call():
    mesh = plsc.VectorSubcoreMesh(core_axis_name="c", subcore_axis_name="s")
    scratch = [pltpu.VMEM((L, D), jnp.float32)]
    scratch += [pltpu.VMEM((1, CHUNK), jnp.int32) for _ in range(NBUF)]
    scratch += [pltpu.VMEM((CHUNK, D), jnp.float32) for _ in range(NBUF)]
    scratch += [pltpu.SemaphoreType.DMA for _ in range(2 * NBUF)]
    return functools.partial(
        pl.kernel,
        out_type=jax.ShapeDtypeStruct((ROWS, D), jnp.float32),
        mesh=mesh,
        scratch_types=scratch,
        compiler_params=pltpu.CompilerParams(use_tc_tiling_on_sc=False),
    )(_sc_kernel)


def kernel(input_sequence, table):
    b, l = input_sequence.shape
    v, d = table.shape
    assert (b, l, d) == (B, L, D)
    idx = input_sequence.astype(jnp.int32).reshape(ROWS // CHUNK, CHUNK)
    pe = _pe_table()
    out = _make_sc_call()(idx, table, pe)
    return out.reshape(B, L, D)


# speedup vs baseline: 1.3560x; 1.3560x over previous
import functools
import math

import jax
import jax.numpy as jnp
from jax import lax
from jax.experimental import pallas as pl
from jax.experimental.pallas import tpu as pltpu
from jax.experimental.pallas import tpu_sc as plsc

D = 64
L = 200
B = 4096
ROWS = B * L
NW = 32
RPW = ROWS // NW
CHUNK = 128
NCHUNK = RPW // CHUNK
NBUF = 4
SCALE = math.sqrt(D)


def _pe_kernel(out_ref):
    pos = lax.broadcasted_iota(jnp.int32, (L, D), 0).astype(jnp.float32)
    col = lax.broadcasted_iota(jnp.int32, (L, D), 1)
    even = col - (col % 2)
    div = jnp.exp(even.astype(jnp.float32) * (-math.log(10000.0) / D))
    ang = pos * div
    out_ref[...] = jnp.where(col % 2 == 0, jnp.sin(ang), jnp.cos(ang))


def _pe_table():
    return pl.pallas_call(
        _pe_kernel,
        out_shape=jax.ShapeDtypeStruct((L, D), jnp.float32),
    )()


def _sc_kernel(idx_hbm, table_hbm, pe_hbm, out_hbm,
               pe_v, idx_all, r0, r1, r2, r3, s0, s1, s2, s3,
               g0, g1, g2, g3, o0, o1, o2, o3):
    rows = [r0, r1, r2, r3]
    stage = [s0, s1, s2, s3]
    gsem = [g0, g1, g2, g3]
    osem = [o0, o1, o2, o3]

    wid = lax.axis_index("s") * 2 + lax.axis_index("c")
    ibase = wid * NCHUNK
    obase = wid * (RPW // 2)

    pltpu.sync_copy(pe_hbm, pe_v)
    pltpu.sync_copy(idx_hbm.at[pl.ds(ibase, NCHUNK)], idx_all)

    def fire(c, b):
        pltpu.async_copy(table_hbm.at[idx_all.at[c]], rows[b], gsem[b])

    def drain_g(b):
        pltpu.make_async_copy(table_hbm.at[pl.ds(0, CHUNK)], rows[b],
                              gsem[b]).wait()

    def drain_o(b):
        pltpu.make_async_copy(out_hbm.at[pl.ds(0, CHUNK // 2)], stage[b],
                              osem[b]).wait()

    for b in range(NBUF - 1):
        fire(b, b)

    def body(g, _):
        for b in range(NBUF):
            c = g * NBUF + b
            bb = (b + NBUF - 1) % NBUF

            if b == 0:
                @pl.when(g > 0)
                def _():
                    drain_o(bb)
                fire(c + NBUF - 1, bb)
            else:
                @pl.when(g < NCHUNK // NBUF - 1)
                def _():
                    drain_o(bb)
                    fire(c + NBUF - 1, bb)

            drain_g(b)

            start = lax.rem(c * CHUNK, L)

            @plsc.parallel_loop(0, CHUNK, unroll=4)
            def crow(r, b=b):
                pos = start + r
                pos = jnp.where(pos >= L, pos - L, pos)
                q = r // 2
                off = (r % 2) * D
                for j in range(D // 16):
                    stage[b][q, pl.ds(off + j * 16, 16)] = (
                        rows[b][r, pl.ds(j * 16, 16)] * SCALE
                        + pe_v[pos, pl.ds(j * 16, 16)])

            pltpu.async_copy(stage[b],
                             out_hbm.at[pl.ds(obase + c * (CHUNK // 2),
                                              CHUNK // 2)],
                             osem[b])
        return ()

    lax.fori_loop(0, NCHUNK // NBUF, body, ())

    for b in range(NBUF):
        drain_o(b)


def _make_sc_call():
    mesh = plsc.VectorSubcoreMesh(core_axis_name="c", subcore_axis_name="s")
    scratch = [pltpu.VMEM((L, D), jnp.float32),
               pltpu.VMEM((NCHUNK, CHUNK), jnp.int32)]
    scratch += [pltpu.VMEM((CHUNK, D), jnp.float32) for _ in range(NBUF)]
    scratch += [pltpu.VMEM((CHUNK // 2, 128), jnp.float32) for _ in range(NBUF)]
    scratch += [pltpu.SemaphoreType.DMA for _ in range(2 * NBUF)]
    return functools.partial(
        pl.kernel,
        out_type=jax.ShapeDtypeStruct((ROWS * D // 128, 128), jnp.float32),
        mesh=mesh,
        scratch_types=scratch,
        compiler_params=pltpu.CompilerParams(use_tc_tiling_on_sc=False),
    )(_sc_kernel)


def kernel(input_sequence, table):
    b, l = input_sequence.shape
    v, d = table.shape
    assert (b, l, d) == (B, L, D)
    idx = input_sequence.astype(jnp.int32).reshape(ROWS // CHUNK, CHUNK)
    pe = _pe_table()
    out = _make_sc_call()(idx, table, pe)
    return out.reshape(B, L, D)
